# baseline (speedup 1.0000x reference)
import jax
import jax.numpy as jnp
from jax import lax
from jax.experimental import pallas as pl
from jax.experimental.pallas import tpu as pltpu
from jax.experimental.pallas import tpu_sc as plsc

N = 128 * 128
K0 = N // 2
NT = 16
CHUNK = N // NT
NV = CHUNK // 16
LEVELS = [(14, 8), (6, 8), (0, 6)]


def _bf16_round(v):
    u = plsc.bitcast(v, jnp.int32)
    r = u + jnp.int32(0x7FFF) + ((u >> 16) & 1)
    return plsc.bitcast(r & jnp.int32(-0x10000), jnp.float32)


def _sc_body(x_hbm, w_hbm, newx_hbm, score_hbm, xv, wv, score_v, p_v,
             out_v, hist_v, ha2, ha3, ha4, merged_v, hs1, hs2, hs3, hs4):
    sid = lax.axis_index("s")
    base = CHUNK * sid
    lanes = jnp.arange(16, dtype=jnp.int32)
    zeros16 = jnp.zeros((16,), jnp.int32)
    ones16 = jnp.ones((16,), jnp.int32)

    xv[pl.ds(0, 16)] = jnp.zeros((16,), jnp.float32)
    xv[pl.ds(CHUNK + 16, 16)] = jnp.zeros((16,), jnp.float32)
    pltpu.sync_copy(x_hbm.at[pl.ds(base, CHUNK)], xv.at[pl.ds(16, CHUNK)])

    @pl.when(sid > 0)
    def _():
        pltpu.sync_copy(x_hbm.at[pl.ds(base - 16, 16)], xv.at[pl.ds(0, 16)])

    @pl.when(sid < NT - 1)
    def _():
        pltpu.sync_copy(
            x_hbm.at[pl.ds(base + CHUNK, 16)], xv.at[pl.ds(CHUNK + 16, 16)]
        )

    pltpu.sync_copy(w_hbm, wv)
    wvec = _bf16_round(wv[pl.ds(0, 16)])
    w = [wvec[d] for d in range(7)]

    def zero_hist(g, _):
        hist_v[pl.ds(16 * g, 16)] = zeros16
        return 0

    lax.fori_loop(0, 16, zero_hist, 0, unroll=4)

    def stage1(j, _):
        o = 16 * j
        t = []
        for d in range(7):
            xb = _bf16_round(xv[pl.ds(o + d + 13, 16)])
            t.append(xb * w[d])
        y = (((t[0] + t[1]) + (t[2] + t[3])) + (t[4] + t[5])) + t[6]
        s = 1.0 / (1.0 + jnp.exp(-y))
        score_v[pl.ds(o, 16)] = s
        p_v[pl.ds(o, 16)] = xv[pl.ds(o + 16, 16)] * (s + 1.0)
        key = plsc.bitcast(s, jnp.int32)
        plsc.addupdate_scatter(hist_v, [key >> 22], ones16)
        return 0

    lax.fori_loop(0, NV, stage1, 0, unroll=4)
    pltpu.sync_copy(score_v, score_hbm.at[pl.ds(base, CHUNK)])
    pltpu.sync_copy(hist_v, hs1.at[sid])
    plsc.subcore_barrier()

    def scan_hist(ngroups, rem):
        def scan(g, carry):
            found, b_sel, cbv, before = carry
            hv = merged_v[pl.ds(16 * g, 16)]
            c = plsc.cumsum(hv)
            tot = c[15]
            ge = (before + c) >= rem
            s = plsc.all_reduce_population_count(ge)
            lane = 16 - s
            hit = jnp.logical_and(jnp.logical_not(found), s > 0)
            cbv = cbv + jnp.where(jnp.logical_and(hit, lanes == lane), c - hv, 0)
            b_sel = jnp.where(hit, 16 * g + lane, b_sel)
            found = jnp.logical_or(found, s > 0)
            return found, b_sel, cbv, before + tot

        init = (jnp.zeros((16,), jnp.bool_), zeros16, zeros16, zeros16)
        _, b_sel, cbv, _ = lax.fori_loop(0, ngroups, scan, init, unroll=4)
        return b_sel, rem - jnp.sum(cbv)

    def merge_into(ha, ngroups):
        def merge(g, _):
            acc = zeros16
            for r in range(NT):
                acc = acc + ha[r, pl.ds(16 * g, 16)]
            merged_v[pl.ds(16 * g, 16)] = acc
            return 0

        lax.fori_loop(0, ngroups, merge, 0, unroll=2)

    pltpu.sync_copy(hs1, ha2)
    merge_into(ha2, 16)
    b_sel, rem = scan_hist(16, jnp.full((16,), K0, jnp.int32))
    prefix = b_sel

    for lvl, (shift, width) in enumerate(LEVELS):
        nb = 1 << width
        ngroups = nb // 16
        hs = (hs2, hs3, hs4)[lvl]
        ha = (ha2, ha3, ha4)[lvl]

        lax.fori_loop(0, ngroups, zero_hist, 0, unroll=4)

        def build(j, _, shift=shift, width=width, prefix=prefix, nb=nb):
            key = plsc.bitcast(score_v[pl.ds(16 * j, 16)], jnp.int32)
            mask = (key >> (shift + width)) == prefix
            b = (key >> shift) & (nb - 1)
            plsc.addupdate_scatter(hist_v, [b], ones16, mask=mask)
            return 0

        lax.fori_loop(0, NV, build, 0, unroll=4)
        pltpu.sync_copy(hist_v.at[pl.ds(0, nb)], hs.at[sid])
        plsc.subcore_barrier()
        pltpu.sync_copy(hs, ha)
        merge_into(ha, ngroups)
        b_sel, rem = scan_hist(ngroups, rem)
        prefix = (prefix << width) | b_sel

    T = prefix
    m = rem

    grp = b_sel[0] >> 4
    lane4 = b_sel & 15
    accv = zeros16
    for t in range(NT):
        row = ha4[t, pl.ds(16 * grp, 16)]
        accv = accv + jnp.where(
            jnp.logical_and(lanes == lane4, jnp.full((16,), t, jnp.int32) < sid),
            row,
            0,
        )
    before_me = jnp.sum(accv)

    def emit(j, r):
        key = plsc.bitcast(score_v[pl.ds(16 * j, 16)], jnp.int32)
        tie = key == T
        inc = plsc.cumsum(tie.astype(jnp.int32))
        sel = jnp.logical_or(key < T, jnp.logical_and(tie, (r + inc) <= m))
        out_v[pl.ds(16 * j, 16)] = jnp.where(sel, p_v[pl.ds(16 * j, 16)], 0.0)
        return r + plsc.all_reduce_population_count(tie)

    lax.fori_loop(0, NV, emit, zeros16 + before_me, unroll=4)
    pltpu.sync_copy(out_v, newx_hbm.at[pl.ds(base, CHUNK)])


def kernel(x, conv_w):
    w16 = jnp.pad(conv_w.reshape(7), (0, 9))
    mesh = plsc.VectorSubcoreMesh(
        core_axis_name="c", subcore_axis_name="s", num_cores=1, num_subcores=16
    )
    newx, score = pl.kernel(
        _sc_body,
        out_type=(
            jax.ShapeDtypeStruct((N,), jnp.float32),
            jax.ShapeDtypeStruct((N,), jnp.float32),
        ),
        mesh=mesh,
        compiler_params=pltpu.CompilerParams(needs_layout_passes=False),
        scratch_types=[
            pltpu.VMEM((CHUNK + 32,), jnp.float32),
            pltpu.VMEM((16,), jnp.float32),
            pltpu.VMEM((CHUNK,), jnp.float32),
            pltpu.VMEM((CHUNK,), jnp.float32),
            pltpu.VMEM((CHUNK,), jnp.float32),
            pltpu.VMEM((256,), jnp.int32),
            pltpu.VMEM((NT, 256), jnp.int32),
            pltpu.VMEM((NT, 256), jnp.int32),
            pltpu.VMEM((NT, 64), jnp.int32),
            pltpu.VMEM((256,), jnp.int32),
            pltpu.VMEM_SHARED((NT, 256), jnp.int32),
            pltpu.VMEM_SHARED((NT, 256), jnp.int32),
            pltpu.VMEM_SHARED((NT, 256), jnp.int32),
            pltpu.VMEM_SHARED((NT, 64), jnp.int32),
        ],
    )(x, w16)
    return newx, score
